# Initial kernel scaffold; baseline (speedup 1.0000x reference)
import jax
import jax.numpy as jnp
from jax.experimental import pallas as pl


def kernel(node_labels, edges, depths, root_ptrs, emb, W_proj, b_proj, W_l, b_l):
    raise NotImplementedError("write your pallas kernel here")


# speedup vs baseline: 3.5697x; 3.5697x over previous
import functools

import jax
import jax.numpy as jnp
from jax import lax
from jax.experimental import pallas as pl
from jax.experimental.pallas import tpu as pltpu
from jax.experimental.pallas import tpu_sc as plsc

f32 = jnp.float32
i32 = jnp.int32

N_NODES = 10000
N_EDGES = 160000
DIM = 256
HALF = 128
NSUB = 16
CHUNK = 128
ZCHUNK = 200
NZCH = N_NODES // ZCHUNK
NZIT = (NZCH + NSUB - 1) // NSUB

_MESH = plsc.VectorSubcoreMesh(core_axis_name="c", subcore_axis_name="s")


_GCHUNKS_FULL = N_NODES // CHUNK
_GTAIL = N_NODES - _GCHUNKS_FULL * CHUNK


@functools.partial(
    pl.kernel,
    out_type=jax.ShapeDtypeStruct((2, N_NODES, HALF), f32),
    mesh=_MESH,
    scratch_types=[
        pltpu.VMEM((CHUNK,), i32),
        pltpu.VMEM((CHUNK, HALF), f32),
        pltpu.VMEM((_GTAIL,), i32),
        pltpu.VMEM((_GTAIL, HALF), f32),
    ],
)
def _sc_embed(table_hbm, idx_hbm, out_hbm, idx_v, rows_v, tidx_v, trows_v):
    c = lax.axis_index("c")
    s = lax.axis_index("s")

    def step(j, carry):
        cid = s + NSUB * j

        @pl.when(cid < _GCHUNKS_FULL)
        def _():
            base = cid * CHUNK
            pltpu.sync_copy(idx_hbm.at[pl.ds(base, CHUNK)], idx_v)
            pltpu.sync_copy(table_hbm.at[c].at[idx_v], rows_v)
            pltpu.sync_copy(rows_v, out_hbm.at[c].at[pl.ds(base, CHUNK)])

        @pl.when(cid == _GCHUNKS_FULL)
        def _():
            base = _GCHUNKS_FULL * CHUNK
            pltpu.sync_copy(idx_hbm.at[pl.ds(base, _GTAIL)], tidx_v)
            pltpu.sync_copy(table_hbm.at[c].at[tidx_v], trows_v)
            pltpu.sync_copy(trows_v, out_hbm.at[c].at[pl.ds(base, _GTAIL)])
        return carry

    lax.fori_loop(0, (_GCHUNKS_FULL + 1 + NSUB - 1) // NSUB, step, 0)


_ECHUNKS = N_EDGES // CHUNK


@functools.partial(
    pl.kernel,
    out_type=jax.ShapeDtypeStruct((2, N_NODES, HALF), f32),
    mesh=_MESH,
    scratch_types=[
        pltpu.VMEM((CHUNK,), i32),
        pltpu.VMEM((CHUNK,), i32),
        pltpu.VMEM((CHUNK, HALF), f32),
        pltpu.VMEM((ZCHUNK, HALF), f32),
        pltpu.VMEM_SHARED((N_NODES, HALF), f32),
    ],
)
def _sc_edge_agg(h_hbm, src_hbm, dst_hbm, zero_hbm, out_hbm,
                 sidx_v, didx_v, rows_v, stage_v, acc_sh):
    c = lax.axis_index("c")
    s = lax.axis_index("s")

    def zstep(j, carry):
        cid = s + NSUB * j

        @pl.when(cid < NZCH)
        def _():
            r0 = cid * ZCHUNK
            pltpu.sync_copy(zero_hbm.at[pl.ds(r0, ZCHUNK)],
                            acc_sh.at[pl.ds(r0, ZCHUNK)])
        return carry

    lax.fori_loop(0, NZIT, zstep, 0)
    plsc.subcore_barrier()

    def step(j, carry):
        cid = s + NSUB * j

        @pl.when(cid < _ECHUNKS)
        def _():
            base = cid * CHUNK
            pltpu.sync_copy(src_hbm.at[pl.ds(base, CHUNK)], sidx_v)
            pltpu.sync_copy(dst_hbm.at[pl.ds(base, CHUNK)], didx_v)
            pltpu.sync_copy(h_hbm.at[c].at[sidx_v], rows_v)
            pltpu.sync_copy(rows_v, acc_sh.at[didx_v], add=True)
        return carry

    lax.fori_loop(0, (_ECHUNKS + NSUB - 1) // NSUB, step, 0)
    plsc.subcore_barrier()

    def wstep(j, carry):
        cid = s + NSUB * j

        @pl.when(cid < NZCH)
        def _():
            r0 = cid * ZCHUNK
            pltpu.sync_copy(acc_sh.at[pl.ds(r0, ZCHUNK)], stage_v)
            pltpu.sync_copy(stage_v, out_hbm.at[c].at[pl.ds(r0, ZCHUNK)])
        return carry

    lax.fori_loop(0, NZIT, wstep, 0)


@functools.partial(
    pl.kernel,
    out_type=jax.ShapeDtypeStruct((2, 16, HALF), f32),
    mesh=_MESH,
    scratch_types=[
        pltpu.VMEM((16,), i32),
        pltpu.VMEM((16, HALF), f32),
    ],
)
def _sc_roots(x_hbm, roots_hbm, out_hbm, idx_v, rows_v):
    c = lax.axis_index("c")
    s = lax.axis_index("s")

    @pl.when(s == 0)
    def _():
        pltpu.sync_copy(roots_hbm, idx_v)
        pltpu.sync_copy(x_hbm.at[c].at[idx_v], rows_v)
        pltpu.sync_copy(rows_v, out_hbm.at[c])


_ROWS_BLK = 1000


def _tc_proj_body(x_ref, w_ref, b_ref, h_ref):
    y = (jnp.dot(x_ref[0], w_ref[:HALF, :], preferred_element_type=f32,
             precision=lax.Precision.HIGHEST)
         + jnp.dot(x_ref[1], w_ref[HALF:, :], preferred_element_type=f32,
             precision=lax.Precision.HIGHEST)
         + b_ref[0])
    y = jnp.maximum(y, 0.0)
    h_ref[0] = y[:, :HALF]
    h_ref[1] = y[:, HALF:]


def _tc_proj(x, w, b):
    return pl.pallas_call(
        _tc_proj_body,
        grid=(N_NODES // _ROWS_BLK,),
        in_specs=[
            pl.BlockSpec((2, _ROWS_BLK, HALF), lambda i: (0, i, 0)),
            pl.BlockSpec((DIM, DIM), lambda i: (0, 0)),
            pl.BlockSpec((1, DIM), lambda i: (0, 0)),
        ],
        out_specs=pl.BlockSpec((2, _ROWS_BLK, HALF), lambda i: (0, i, 0)),
        out_shape=jax.ShapeDtypeStruct((2, N_NODES, HALF), f32),
    )(x, w, b)


def _tc_lin_body(a_ref, degp_ref, w_ref, b_ref, x_ref):
    deg = degp_ref[0, :, 0]
    scale = 1.0 / jnp.maximum(deg, 1.0)
    a0 = a_ref[0] * scale[:, None]
    a1 = a_ref[1] * scale[:, None]
    y = (jnp.dot(a0, w_ref[:HALF, :], preferred_element_type=f32,
             precision=lax.Precision.HIGHEST)
         + jnp.dot(a1, w_ref[HALF:, :], preferred_element_type=f32,
             precision=lax.Precision.HIGHEST)
         + b_ref[0])
    x_ref[0] = y[:, :HALF]
    x_ref[1] = y[:, HALF:]


def _tc_lin(a, degp, w, b):
    return pl.pallas_call(
        _tc_lin_body,
        grid=(N_NODES // _ROWS_BLK,),
        in_specs=[
            pl.BlockSpec((2, _ROWS_BLK, HALF), lambda i: (0, i, 0)),
            pl.BlockSpec((2, _ROWS_BLK, HALF), lambda i: (0, i, 0)),
            pl.BlockSpec((DIM, DIM), lambda i: (0, 0)),
            pl.BlockSpec((1, DIM), lambda i: (0, 0)),
        ],
        out_specs=pl.BlockSpec((2, _ROWS_BLK, HALF), lambda i: (0, i, 0)),
        out_shape=jax.ShapeDtypeStruct((2, N_NODES, HALF), f32),
    )(a, degp, w, b)


def kernel(node_labels, edges, depths, root_ptrs, emb, W_proj, b_proj, W_l, b_l):
    labels = node_labels.astype(i32)
    src = edges[0].astype(i32)
    dst = edges[1].astype(i32)
    roots = (root_ptrs[1:] - 1).astype(i32)

    emb2 = emb.reshape(-1, 2, HALF).transpose(1, 0, 2)
    zeros_nd = jnp.zeros((N_NODES, HALF), f32)
    b_proj2 = b_proj.reshape(1, DIM)
    b_l2 = b_l.reshape(1, DIM)

    ones_nd = jnp.ones((2, N_NODES, HALF), f32)
    degp = _sc_edge_agg(ones_nd, src, dst, zeros_nd)
    x = _sc_embed(emb2, labels)
    depth = jnp.max(depths)

    def body(_, x):
        h = _tc_proj(x, W_proj, b_proj2)
        agg = _sc_edge_agg(h, src, dst, zeros_nd)
        return _tc_lin(agg, degp, W_l, b_l2)

    x = lax.fori_loop(0, depth, body, x)
    o = _sc_roots(x, roots)
    return jnp.concatenate([o[0], o[1]], axis=-1)
